# Initial kernel scaffold; baseline (speedup 1.0000x reference)
import jax
import jax.numpy as jnp
from jax.experimental import pallas as pl


def kernel(feat_ref, flow, W1, b1, W2, b2):
    raise NotImplementedError("write your pallas kernel here")


# speedup vs baseline: 1.9375x; 1.9375x over previous
import functools

import jax
import jax.numpy as jnp
from jax import lax
from jax.experimental import pallas as pl
from jax.experimental.pallas import tpu as pltpu
from jax.experimental.pallas import tpu_sc as plsc

B, C, H, W = 4, 128, 224, 224
HW = H * W
CO1 = 64
RB = 32
NB = H // RB

NCORES, NSUB, LANES = 2, 16, 16
NWORKERS = NCORES * NSUB
NCHUNK = C // 2 + 1
NTASK = B * NCHUNK
NSLOT = (NTASK + NWORKERS - 1) // NWORKERS
S = 1792
NSLICES = HW // S
GROUPS = S // LANES


def _metric_body(feat_hbm, w1_ref, b1_ref, w2_ref, b2_ref, out_ref,
                 h_pad, in_buf, sem):
    bb = pl.program_id(0)
    i = pl.program_id(1)
    g = bb * NB + i

    @pl.when(i == 0)
    def _zero():
        h_pad[...] = jnp.zeros((H + 2, W + 2, CO1), jnp.float32)

    def copy_band(gg, k):
        bn = lax.div(gg, NB)
        bi = gg - bn * NB
        return pltpu.make_async_copy(
            feat_hbm.at[bn, pl.ds(bi * RB, RB + 2)], in_buf.at[k], sem.at[k])

    @pl.when(jnp.logical_and(bb == 0, i == 0))
    def _prime():
        copy_band(0, 0).start()

    @pl.when(i < NB)
    def _conv1():
        k = lax.rem(g, 2)
        copy_band(g, k).wait()

        @pl.when(g + 1 < B * NB)
        def _prefetch():
            copy_band(g + 1, 1 - k).start()

        acc = jnp.zeros((RB * W, CO1), jnp.float32)
        for ky in range(3):
            for kx in range(3):
                x = in_buf[k, pl.ds(ky, RB), pl.ds(kx, W), :]
                acc += jnp.dot(x.reshape(RB * W, C), w1_ref[3 * ky + kx],
                               preferred_element_type=jnp.float32)
        acc += b1_ref[...]
        h = acc * jax.nn.sigmoid(acc)
        h_pad[pl.ds(i * RB + 1, RB), pl.ds(1, W), :] = h.reshape(RB, W, CO1)

    @pl.when(i >= 1)
    def _conv2():
        y0 = (i - 1) * RB
        macc = jnp.zeros((RB * W, 8), jnp.float32)
        for ky in range(3):
            for kx in range(3):
                xs = h_pad[pl.ds(y0 + ky, RB), pl.ds(kx, W), :]
                macc += jnp.dot(xs.reshape(RB * W, CO1), w2_ref[3 * ky + kx],
                                preferred_element_type=jnp.float32)
        out_ref[0] = (macc + b2_ref[...]).reshape(RB, W, 8)


def _run_metric(featp, w1r, b1r, w2r, b2r):
    return pl.pallas_call(
        _metric_body,
        grid=(B, NB + 1),
        in_specs=[
            pl.BlockSpec(memory_space=pl.ANY),
            pl.BlockSpec((9, C, CO1), lambda b, i: (0, 0, 0)),
            pl.BlockSpec((1, CO1), lambda b, i: (0, 0)),
            pl.BlockSpec((9, CO1, 8), lambda b, i: (0, 0, 0)),
            pl.BlockSpec((1, 8), lambda b, i: (0, 0)),
        ],
        out_specs=pl.BlockSpec(
            (1, RB, W, 8),
            lambda b, i: (b, jnp.clip(i - 1, 0, NB - 1), 0, 0)),
        out_shape=jax.ShapeDtypeStruct((B, H, W, 8), jnp.float32),
        scratch_shapes=[
            pltpu.VMEM((H + 2, W + 2, CO1), jnp.float32),
            pltpu.VMEM((2, RB + 2, W + 2, C), jnp.float32),
            pltpu.SemaphoreType.DMA((2,)),
        ],
    )(featp, w1r, b1r, w2r, b2r)


def _taps_body(metric_ref, flow_ref, wm_ref, idx_ref):
    xg = lax.broadcasted_iota(jnp.int32, (H, W), 1).astype(jnp.float32)
    yg = lax.broadcasted_iota(jnp.int32, (H, W), 0).astype(jnp.float32)
    gx = xg + flow_ref[0, 0]
    gy = yg + flow_ref[0, 1]
    x0 = jnp.floor(gx)
    y0 = jnp.floor(gy)
    m = jnp.exp(metric_ref[0])
    t = 0
    for dy in (0.0, 1.0):
        for dx in (0.0, 1.0):
            xi = x0 + dx
            yi = y0 + dy
            w = (1.0 - jnp.abs(gx - xi)) * (1.0 - jnp.abs(gy - yi))
            valid = (xi >= 0.0) & (xi <= W - 1.0) & (yi >= 0.0) & (yi <= H - 1.0)
            wv = jnp.where(valid, w, 0.0) * m
            xc = jnp.clip(xi, 0.0, W - 1.0).astype(jnp.int32)
            yc = jnp.clip(yi, 0.0, H - 1.0).astype(jnp.int32)
            idx = jnp.clip(yc * W + xc, 0, HW - 1)
            wm_ref[0, t] = wv
            idx_ref[0, t] = idx
            t += 1


def _run_taps(metric2d, flow):
    return pl.pallas_call(
        _taps_body,
        grid=(B,),
        in_specs=[
            pl.BlockSpec((1, H, W), lambda b: (b, 0, 0)),
            pl.BlockSpec((1, 2, H, W), lambda b: (b, 0, 0, 0)),
        ],
        out_specs=[
            pl.BlockSpec((1, 4, H, W), lambda b: (b, 0, 0, 0)),
            pl.BlockSpec((1, 4, H, W), lambda b: (b, 0, 0, 0)),
        ],
        out_shape=[
            jax.ShapeDtypeStruct((B, 4, H, W), jnp.float32),
            jax.ShapeDtypeStruct((B, 4, H, W), jnp.int32),
        ],
    )(metric2d, flow)


def _splat_body(wm_hbm, idx_hbm, feat_hbm, out_hbm, acc, wv, iv, vl):
    wid = lax.axis_index("s") * NCORES + lax.axis_index("c")
    for slot in range(NSLOT):
        task = slot * NWORKERS + wid

        @pl.when(task < NTASK)
        def _():
            b = jnp.bitwise_and(task, 3)
            chunk = jnp.right_shift(task, 2)

            def zero_body(i, _):
                for k in range(8):
                    acc[pl.ds(i * 128 + k * 16, 16)] = jnp.zeros((16,), jnp.float32)
                return 0

            lax.fori_loop(0, 2 * HW // 128, zero_body, 0)

            @pl.when(chunk < NCHUNK - 1)
            def _feat():
                def slice_body(si, _):
                    off = si * S
                    pltpu.sync_copy(wm_hbm.at[b, :, pl.ds(off, S)], wv)
                    pltpu.sync_copy(idx_hbm.at[b, :, pl.ds(off, S)], iv)
                    pltpu.sync_copy(feat_hbm.at[b, pl.ds(chunk * 2, 2), pl.ds(off, S)], vl)

                    def grp(g, _):
                        g0 = g * LANES
                        v0 = vl[0, pl.ds(g0, LANES)]
                        v1 = vl[1, pl.ds(g0, LANES)]
                        for t in range(4):
                            w = wv[t, pl.ds(g0, LANES)]
                            ix = iv[t, pl.ds(g0, LANES)]
                            plsc.addupdate_scatter(acc, [ix], v0 * w)
                            plsc.addupdate_scatter(acc, [ix + HW], v1 * w)
                        return 0

                    lax.fori_loop(0, GROUPS, grp, 0)
                    return 0

                lax.fori_loop(0, NSLICES, slice_body, 0)
                base = (b * (C + 1) + chunk * 2) * HW
                pltpu.sync_copy(acc.at[pl.ds(0, HW)], out_hbm.at[pl.ds(base, HW)])
                pltpu.sync_copy(acc.at[pl.ds(HW, HW)], out_hbm.at[pl.ds(base + HW, HW)])

            @pl.when(chunk == NCHUNK - 1)
            def _dens():
                def slice_body(si, _):
                    off = si * S
                    pltpu.sync_copy(wm_hbm.at[b, :, pl.ds(off, S)], wv)
                    pltpu.sync_copy(idx_hbm.at[b, :, pl.ds(off, S)], iv)

                    def grp(g, _):
                        g0 = g * LANES
                        for t in range(4):
                            w = wv[t, pl.ds(g0, LANES)]
                            ix = iv[t, pl.ds(g0, LANES)]
                            plsc.addupdate_scatter(acc, [ix], w)
                        return 0

                    lax.fori_loop(0, GROUPS, grp, 0)
                    return 0

                lax.fori_loop(0, NSLICES, slice_body, 0)
                base = (b * (C + 1) + C) * HW
                pltpu.sync_copy(acc.at[pl.ds(0, HW)], out_hbm.at[pl.ds(base, HW)])


def _run_splat(wm2, idx2, feat2):
    mesh = plsc.VectorSubcoreMesh(core_axis_name="c", subcore_axis_name="s",
                                  num_cores=NCORES, num_subcores=NSUB)
    f = functools.partial(
        pl.kernel,
        out_type=jax.ShapeDtypeStruct((B * (C + 1) * HW,), jnp.float32),
        mesh=mesh,
        compiler_params=pltpu.CompilerParams(needs_layout_passes=False),
        scratch_types=[
            pltpu.VMEM((2 * HW,), jnp.float32),
            pltpu.VMEM((4, S), jnp.float32),
            pltpu.VMEM((4, S), jnp.int32),
            pltpu.VMEM((2, S), jnp.float32),
        ],
    )(_splat_body)
    return f(wm2, idx2, feat2)


HWD = HW // 8


def _norm_body(splat_ref, out_ref):
    s = splat_ref[0]
    out_ref[0] = s[0:C] / jnp.maximum(s[C:C + 1], 1e-7)


def _run_norm(splat):
    return pl.pallas_call(
        _norm_body,
        grid=(B, 8),
        in_specs=[pl.BlockSpec((1, C + 1, HWD), lambda b, j: (b, 0, j))],
        out_specs=pl.BlockSpec((1, C, HWD), lambda b, j: (b, 0, j)),
        out_shape=jax.ShapeDtypeStruct((B, C, HW), jnp.float32),
    )(splat)


def kernel(feat_ref, flow, W1, b1, W2, b2):
    featp = jnp.pad(jnp.transpose(feat_ref, (0, 2, 3, 1)),
                    ((0, 0), (1, 1), (1, 1), (0, 0)))
    w1r = jnp.transpose(W1, (2, 3, 1, 0)).reshape(9, C, CO1)
    b1r = b1.reshape(1, CO1)
    w2r = jnp.pad(jnp.transpose(W2, (2, 3, 1, 0)).reshape(9, CO1, 1),
                  ((0, 0), (0, 0), (0, 7)))
    b2r = jnp.pad(b2.reshape(1, 1), ((0, 0), (0, 7)))

    metric8 = _run_metric(featp, w1r, b1r, w2r, b2r)
    metric2d = metric8[..., 0]

    wm, idxo = _run_taps(metric2d, flow)
    feat2 = feat_ref.reshape(B, C, HW)

    splat = _run_splat(wm.reshape(B, 4, HW), idxo.reshape(B, 4, HW),
                       feat2).reshape(B, C + 1, HW)
    warped = _run_norm(splat).reshape(B, C, H, W)
    return warped, metric2d[:, None]


# speedup vs baseline: 2.1658x; 1.1178x over previous
import functools

import jax
import jax.numpy as jnp
from jax import lax
from jax.experimental import pallas as pl
from jax.experimental.pallas import tpu as pltpu
from jax.experimental.pallas import tpu_sc as plsc

B, C, H, W = 4, 128, 224, 224
HW = H * W
CO1 = 64
RB = 32
NB = H // RB

NCORES, NSUB, LANES = 2, 16, 16
NWORKERS = NCORES * NSUB
NCHUNK = C // 2 + 1
NTASK = B * NCHUNK
NSLOT = (NTASK + NWORKERS - 1) // NWORKERS
S = 1024
NSLICES = HW // S
GROUPS = S // LANES


def _metric_body(feat_hbm, w1_ref, b1_ref, w2_ref, b2_ref, out_ref,
                 h_pad, in_buf, sem):
    bb = pl.program_id(0)
    i = pl.program_id(1)
    g = bb * NB + i

    @pl.when(i == 0)
    def _zero():
        h_pad[...] = jnp.zeros((H + 2, W + 2, CO1), jnp.float32)

    def copy_band(gg, k):
        bn = lax.div(gg, NB)
        bi = gg - bn * NB
        return pltpu.make_async_copy(
            feat_hbm.at[bn, pl.ds(bi * RB, RB + 2)], in_buf.at[k], sem.at[k])

    @pl.when(jnp.logical_and(bb == 0, i == 0))
    def _prime():
        copy_band(0, 0).start()

    @pl.when(i < NB)
    def _conv1():
        k = lax.rem(g, 2)
        copy_band(g, k).wait()

        @pl.when(g + 1 < B * NB)
        def _prefetch():
            copy_band(g + 1, 1 - k).start()

        acc = jnp.zeros((RB * W, CO1), jnp.float32)
        for ky in range(3):
            for kx in range(3):
                x = in_buf[k, pl.ds(ky, RB), pl.ds(kx, W), :]
                acc += jnp.dot(x.reshape(RB * W, C), w1_ref[3 * ky + kx],
                               preferred_element_type=jnp.float32)
        acc += b1_ref[...]
        h = acc * jax.nn.sigmoid(acc)
        h_pad[pl.ds(i * RB + 1, RB), pl.ds(1, W), :] = h.reshape(RB, W, CO1)

    @pl.when(i >= 1)
    def _conv2():
        y0 = (i - 1) * RB
        macc = jnp.zeros((RB * W, 8), jnp.float32)
        for ky in range(3):
            for kx in range(3):
                xs = h_pad[pl.ds(y0 + ky, RB), pl.ds(kx, W), :]
                macc += jnp.dot(xs.reshape(RB * W, CO1), w2_ref[3 * ky + kx],
                                preferred_element_type=jnp.float32)
        out_ref[0] = (macc + b2_ref[...]).reshape(RB, W, 8)


def _run_metric(featp, w1r, b1r, w2r, b2r):
    return pl.pallas_call(
        _metric_body,
        grid=(B, NB + 1),
        in_specs=[
            pl.BlockSpec(memory_space=pl.ANY),
            pl.BlockSpec((9, C, CO1), lambda b, i: (0, 0, 0)),
            pl.BlockSpec((1, CO1), lambda b, i: (0, 0)),
            pl.BlockSpec((9, CO1, 8), lambda b, i: (0, 0, 0)),
            pl.BlockSpec((1, 8), lambda b, i: (0, 0)),
        ],
        out_specs=pl.BlockSpec(
            (1, RB, W, 8),
            lambda b, i: (b, jnp.clip(i - 1, 0, NB - 1), 0, 0)),
        out_shape=jax.ShapeDtypeStruct((B, H, W, 8), jnp.float32),
        scratch_shapes=[
            pltpu.VMEM((H + 2, W + 2, CO1), jnp.float32),
            pltpu.VMEM((2, RB + 2, W + 2, C), jnp.float32),
            pltpu.SemaphoreType.DMA((2,)),
        ],
    )(featp, w1r, b1r, w2r, b2r)


def _taps_body(metric_ref, flow_ref, wm_ref, idx_ref):
    xg = lax.broadcasted_iota(jnp.int32, (H, W), 1).astype(jnp.float32)
    yg = lax.broadcasted_iota(jnp.int32, (H, W), 0).astype(jnp.float32)
    gx = xg + flow_ref[0, 0]
    gy = yg + flow_ref[0, 1]
    x0 = jnp.floor(gx)
    y0 = jnp.floor(gy)
    m = jnp.exp(metric_ref[0])
    t = 0
    for dy in (0.0, 1.0):
        for dx in (0.0, 1.0):
            xi = x0 + dx
            yi = y0 + dy
            w = (1.0 - jnp.abs(gx - xi)) * (1.0 - jnp.abs(gy - yi))
            valid = (xi >= 0.0) & (xi <= W - 1.0) & (yi >= 0.0) & (yi <= H - 1.0)
            wv = jnp.where(valid, w, 0.0) * m
            xc = jnp.clip(xi, 0.0, W - 1.0).astype(jnp.int32)
            yc = jnp.clip(yi, 0.0, H - 1.0).astype(jnp.int32)
            idx = jnp.clip(yc * W + xc, 0, HW - 1).astype(jnp.float32)
            wm_ref[0, t] = wv
            idx_ref[0, t] = idx
            t += 1


def _run_taps(metric2d, flow):
    return pl.pallas_call(
        _taps_body,
        grid=(B,),
        in_specs=[
            pl.BlockSpec((1, H, W), lambda b: (b, 0, 0)),
            pl.BlockSpec((1, 2, H, W), lambda b: (b, 0, 0, 0)),
        ],
        out_specs=[
            pl.BlockSpec((1, 4, H, W), lambda b: (b, 0, 0, 0)),
            pl.BlockSpec((1, 4, H, W), lambda b: (b, 0, 0, 0)),
        ],
        out_shape=[
            jax.ShapeDtypeStruct((B, 4, H, W), jnp.float32),
            jax.ShapeDtypeStruct((B, 4, H, W), jnp.float32),
        ],
    )(metric2d, flow)


def _splat_body(pk_hbm, feat_hbm, out_hbm, acc, pk, vl, sems):
    wid = lax.axis_index("s") * NCORES + lax.axis_index("c")

    def issue(b, chunk, si, k, with_vals):
        off = si * S
        pltpu.async_copy(pk_hbm.at[b, :, pl.ds(off, S)], pk.at[k], sems.at[k])
        if with_vals:
            pltpu.async_copy(
                feat_hbm.at[b, pl.ds(chunk * 2, 2), pl.ds(off, S)],
                vl.at[k], sems.at[k])

    def drain(b, k, with_vals):
        pltpu.make_async_copy(pk_hbm.at[b, :, pl.ds(0, S)], pk.at[k],
                              sems.at[k]).wait()
        if with_vals:
            pltpu.make_async_copy(feat_hbm.at[b, pl.ds(0, 2), pl.ds(0, S)],
                                  vl.at[k], sems.at[k]).wait()

    def zero_acc(nwords):
        def zero_body(i, _):
            for j in range(8):
                acc[pl.ds(i * 128 + j * 16, 16)] = jnp.zeros((16,), jnp.float32)
            return 0

        lax.fori_loop(0, nwords // 128, zero_body, 0)

    def do_feat(b, chunk):
        issue(b, chunk, 0, 0, True)
        zero_acc(2 * HW)

        def slice_body(si, _):
            k = lax.rem(si, 2)

            @pl.when(si + 1 < NSLICES)
            def _pref():
                issue(b, chunk, si + 1, 1 - k, True)

            drain(b, k, True)

            def grp(g, _):
                g0 = g * LANES
                v0 = vl[k, 0, pl.ds(g0, LANES)]
                v1 = vl[k, 1, pl.ds(g0, LANES)]
                for t in range(4):
                    w = pk[k, t, pl.ds(g0, LANES)]
                    ix = pk[k, 4 + t, pl.ds(g0, LANES)].astype(jnp.int32)
                    plsc.addupdate_scatter(acc, [ix], v0 * w)
                    plsc.addupdate_scatter(acc, [ix + HW], v1 * w)
                return 0

            lax.fori_loop(0, GROUPS, grp, 0)
            return 0

        lax.fori_loop(0, NSLICES, slice_body, 0)
        base = (b * (C + 1) + chunk * 2) * HW
        pltpu.sync_copy(acc.at[pl.ds(0, HW)], out_hbm.at[pl.ds(base, HW)])
        pltpu.sync_copy(acc.at[pl.ds(HW, HW)], out_hbm.at[pl.ds(base + HW, HW)])

    def do_dens(b):
        issue(b, 0, 0, 0, False)
        zero_acc(HW)

        def slice_body(si, _):
            k = lax.rem(si, 2)

            @pl.when(si + 1 < NSLICES)
            def _pref():
                issue(b, 0, si + 1, 1 - k, False)

            drain(b, k, False)

            def grp(g, _):
                g0 = g * LANES
                for t in range(4):
                    w = pk[k, t, pl.ds(g0, LANES)]
                    ix = pk[k, 4 + t, pl.ds(g0, LANES)].astype(jnp.int32)
                    plsc.addupdate_scatter(acc, [ix], w)
                return 0

            lax.fori_loop(0, GROUPS, grp, 0)
            return 0

        lax.fori_loop(0, NSLICES, slice_body, 0)
        base = (b * (C + 1) + C) * HW
        pltpu.sync_copy(acc.at[pl.ds(0, HW)], out_hbm.at[pl.ds(base, HW)])

    for slot in range(8):
        task = slot * NWORKERS + wid
        b = jnp.bitwise_and(task, 3)
        pair = jnp.right_shift(task, 2)
        do_feat(b, pair)

    @pl.when(wid < B)
    def _dens():
        do_dens(wid)


def _run_splat(packed8, feat2):
    mesh = plsc.VectorSubcoreMesh(core_axis_name="c", subcore_axis_name="s",
                                  num_cores=NCORES, num_subcores=NSUB)
    f = functools.partial(
        pl.kernel,
        out_type=jax.ShapeDtypeStruct((B * (C + 1) * HW,), jnp.float32),
        mesh=mesh,
        compiler_params=pltpu.CompilerParams(needs_layout_passes=False),
        scratch_types=[
            pltpu.VMEM((2 * HW,), jnp.float32),
            pltpu.VMEM((2, 8, S), jnp.float32),
            pltpu.VMEM((2, 2, S), jnp.float32),
            pltpu.SemaphoreType.DMA((2,)),
        ],
    )(_splat_body)
    return f(packed8, feat2)


HWD = HW // 8


def _norm_body(splat_ref, out_ref):
    s = splat_ref[0]
    out_ref[0] = s[0:C] / jnp.maximum(s[C:C + 1], 1e-7)


def _run_norm(splat):
    return pl.pallas_call(
        _norm_body,
        grid=(B, 8),
        in_specs=[pl.BlockSpec((1, C + 1, HWD), lambda b, j: (b, 0, j))],
        out_specs=pl.BlockSpec((1, C, HWD), lambda b, j: (b, 0, j)),
        out_shape=jax.ShapeDtypeStruct((B, C, HW), jnp.float32),
    )(splat)


def kernel(feat_ref, flow, W1, b1, W2, b2):
    featp = jnp.pad(jnp.transpose(feat_ref, (0, 2, 3, 1)),
                    ((0, 0), (1, 1), (1, 1), (0, 0)))
    w1r = jnp.transpose(W1, (2, 3, 1, 0)).reshape(9, C, CO1)
    b1r = b1.reshape(1, CO1)
    w2r = jnp.pad(jnp.transpose(W2, (2, 3, 1, 0)).reshape(9, CO1, 1),
                  ((0, 0), (0, 0), (0, 7)))
    b2r = jnp.pad(b2.reshape(1, 1), ((0, 0), (0, 7)))

    metric8 = _run_metric(featp, w1r, b1r, w2r, b2r)
    metric2d = metric8[..., 0]

    wm, idxo = _run_taps(metric2d, flow)
    packed8 = jnp.concatenate(
        [wm.reshape(B, 4, HW), idxo.reshape(B, 4, HW)], axis=1)
    feat2 = feat_ref.reshape(B, C, HW)

    splat = _run_splat(packed8, feat2).reshape(B, C + 1, HW)
    warped = _run_norm(splat).reshape(B, C, H, W)
    return warped, metric2d[:, None]


# speedup vs baseline: 3.0863x; 1.4250x over previous
import functools

import jax
import jax.numpy as jnp
from jax import lax
from jax.experimental import pallas as pl
from jax.experimental.pallas import tpu as pltpu
from jax.experimental.pallas import tpu_sc as plsc

B, C, H, W = 4, 128, 224, 224
HW = H * W
CO1 = 64
RB = 32
NB = H // RB

NCORES, NSUB, LANES = 2, 16, 16
NWORKERS = NCORES * NSUB
NCHUNK = C // 2 + 1
NTASK = B * NCHUNK
NSLOT = (NTASK + NWORKERS - 1) // NWORKERS
S = 1024
NSLICES = HW // S
GROUPS = S // LANES


def _metric_body(feat_hbm, w1_ref, b1_ref, w2_ref, b2_ref, out_ref,
                 h_pad, in_buf, sem):
    bb = pl.program_id(0)
    i = pl.program_id(1)
    g = bb * NB + i

    @pl.when(i == 0)
    def _zero():
        h_pad[...] = jnp.zeros((H + 2, W + 2, CO1), jnp.float32)

    def copy_band(gg, k):
        bn = lax.div(gg, NB)
        bi = gg - bn * NB
        return pltpu.make_async_copy(
            feat_hbm.at[bn, pl.ds(bi * RB, RB + 2)], in_buf.at[k], sem.at[k])

    @pl.when(jnp.logical_and(bb == 0, i == 0))
    def _prime():
        copy_band(0, 0).start()

    @pl.when(i < NB)
    def _conv1():
        k = lax.rem(g, 2)
        copy_band(g, k).wait()

        @pl.when(g + 1 < pl.num_programs(0) * NB)
        def _prefetch():
            copy_band(g + 1, 1 - k).start()

        acc = jnp.zeros((RB * W, CO1), jnp.float32)
        for ky in range(3):
            for kx in range(3):
                x = in_buf[k, pl.ds(ky, RB), pl.ds(kx, W), :]
                acc += jnp.dot(x.reshape(RB * W, C), w1_ref[3 * ky + kx],
                               preferred_element_type=jnp.float32)
        acc += b1_ref[...]
        h = acc * jax.nn.sigmoid(acc)
        h_pad[pl.ds(i * RB + 1, RB), pl.ds(1, W), :] = h.reshape(RB, W, CO1)

    @pl.when(i >= 1)
    def _conv2():
        y0 = (i - 1) * RB
        macc = jnp.zeros((RB * W, 8), jnp.float32)
        for ky in range(3):
            for kx in range(3):
                xs = h_pad[pl.ds(y0 + ky, RB), pl.ds(kx, W), :]
                macc += jnp.dot(xs.reshape(RB * W, CO1), w2_ref[3 * ky + kx],
                                preferred_element_type=jnp.float32)
        out_ref[0] = (macc + b2_ref[...]).reshape(RB, W, 8)


def _run_metric(featp, w1r, b1r, w2r, b2r):
    Bk = featp.shape[0]
    return pl.pallas_call(
        _metric_body,
        grid=(Bk, NB + 1),
        in_specs=[
            pl.BlockSpec(memory_space=pl.ANY),
            pl.BlockSpec((9, C, CO1), lambda b, i: (0, 0, 0)),
            pl.BlockSpec((1, CO1), lambda b, i: (0, 0)),
            pl.BlockSpec((9, CO1, 8), lambda b, i: (0, 0, 0)),
            pl.BlockSpec((1, 8), lambda b, i: (0, 0)),
        ],
        out_specs=pl.BlockSpec(
            (1, RB, W, 8),
            lambda b, i: (b, jnp.clip(i - 1, 0, NB - 1), 0, 0)),
        out_shape=jax.ShapeDtypeStruct((Bk, H, W, 8), jnp.float32),
        scratch_shapes=[
            pltpu.VMEM((H + 2, W + 2, CO1), jnp.float32),
            pltpu.VMEM((2, RB + 2, W + 2, C), jnp.float32),
            pltpu.SemaphoreType.DMA((2,)),
        ],
    )(featp, w1r, b1r, w2r, b2r)


def _taps_body(metric_ref, flow_ref, wm_ref, idx_ref):
    xg = lax.broadcasted_iota(jnp.int32, (H, W), 1).astype(jnp.float32)
    yg = lax.broadcasted_iota(jnp.int32, (H, W), 0).astype(jnp.float32)
    gx = xg + flow_ref[0, 0]
    gy = yg + flow_ref[0, 1]
    x0 = jnp.floor(gx)
    y0 = jnp.floor(gy)
    m = jnp.exp(metric_ref[0])
    t = 0
    for dy in (0.0, 1.0):
        for dx in (0.0, 1.0):
            xi = x0 + dx
            yi = y0 + dy
            w = (1.0 - jnp.abs(gx - xi)) * (1.0 - jnp.abs(gy - yi))
            valid = (xi >= 0.0) & (xi <= W - 1.0) & (yi >= 0.0) & (yi <= H - 1.0)
            wv = jnp.where(valid, w, 0.0) * m
            xc = jnp.clip(xi, 0.0, W - 1.0).astype(jnp.int32)
            yc = jnp.clip(yi, 0.0, H - 1.0).astype(jnp.int32)
            idx = jnp.clip(yc * W + xc, 0, HW - 1).astype(jnp.float32)
            wm_ref[0, t] = wv
            idx_ref[0, t] = idx
            t += 1


def _run_taps(metric2d, flow):
    Bk = metric2d.shape[0]
    return pl.pallas_call(
        _taps_body,
        grid=(Bk,),
        in_specs=[
            pl.BlockSpec((1, H, W), lambda b: (b, 0, 0)),
            pl.BlockSpec((1, 2, H, W), lambda b: (b, 0, 0, 0)),
        ],
        out_specs=[
            pl.BlockSpec((1, 4, H, W), lambda b: (b, 0, 0, 0)),
            pl.BlockSpec((1, 4, H, W), lambda b: (b, 0, 0, 0)),
        ],
        out_shape=[
            jax.ShapeDtypeStruct((Bk, 4, H, W), jnp.float32),
            jax.ShapeDtypeStruct((Bk, 4, H, W), jnp.float32),
        ],
    )(metric2d, flow)


NROW = C + 4
QSL = NSLICES // 4


def _splat_body(pk_hbm, feat_hbm, out_hbm, acc0, acc1, pk, vl, sems):
    wid = lax.axis_index("s") * NCORES + lax.axis_index("c")

    def issue(chunk, si, k, with_vals):
        off = si * S
        pltpu.async_copy(pk_hbm.at[0, :, pl.ds(off, S)], pk.at[k], sems.at[k])
        if with_vals:
            pltpu.async_copy(
                feat_hbm.at[0, pl.ds(chunk * 2, 2), pl.ds(off, S)],
                vl.at[k], sems.at[k])

    def drain(k, with_vals):
        pltpu.make_async_copy(pk_hbm.at[0, :, pl.ds(0, S)], pk.at[k],
                              sems.at[k]).wait()
        if with_vals:
            pltpu.make_async_copy(feat_hbm.at[0, pl.ds(0, 2), pl.ds(0, S)],
                                  vl.at[k], sems.at[k]).wait()

    def zero_acc(both):
        def _zero(i, _):
            for j in range(4):
                acc0[pl.ds(i * 64 + j * 16, 16)] = jnp.zeros((16,), jnp.float32)
                if both:
                    acc1[pl.ds(i * 64 + j * 16, 16)] = jnp.zeros((16,),
                                                                 jnp.float32)
            return 0

        lax.fori_loop(0, HW // 64, _zero, 0)

    def do_feat(chunk):
        issue(chunk, 0, 0, True)
        zero_acc(True)

        def slice_body(si, _):
            k = lax.rem(si, 2)

            @pl.when(si + 1 < NSLICES)
            def _pref():
                issue(chunk, si + 1, 1 - k, True)

            drain(k, True)

            def grp(g, _):
                g0 = g * LANES
                v0 = vl[k, 0, pl.ds(g0, LANES)]
                v1 = vl[k, 1, pl.ds(g0, LANES)]
                for t in range(4):
                    w = pk[k, t, pl.ds(g0, LANES)]
                    ix = pk[k, 4 + t, pl.ds(g0, LANES)].astype(jnp.int32)
                    plsc.addupdate_scatter(acc0, [ix], v0 * w)
                    plsc.addupdate_scatter(acc1, [ix], v1 * w)
                return 0

            lax.fori_loop(0, GROUPS, grp, 0)
            return 0

        lax.fori_loop(0, NSLICES, slice_body, 0)
        base = chunk * 2 * HW
        pltpu.sync_copy(acc0, out_hbm.at[pl.ds(base, HW)])
        pltpu.sync_copy(acc1, out_hbm.at[pl.ds(base + HW, HW)])

    def do_dens(q):
        s_lo = q * QSL
        s_hi = jnp.where(q == 3, NSLICES, s_lo + QSL)
        issue(0, s_lo, 0, False)
        zero_acc(True)

        def slice_body(si, _):
            k = lax.rem(si - s_lo, 2)

            @pl.when(si + 1 < s_hi)
            def _pref():
                issue(0, si + 1, 1 - k, False)

            drain(k, False)

            def grp(g, _):
                g0 = g * LANES
                for t in range(4):
                    w = pk[k, t, pl.ds(g0, LANES)]
                    ix = pk[k, 4 + t, pl.ds(g0, LANES)].astype(jnp.int32)
                    plsc.addupdate_scatter(acc0 if t < 2 else acc1, [ix], w)
                return 0

            lax.fori_loop(0, GROUPS, grp, 0)
            return 0

        lax.fori_loop(s_lo, s_hi, slice_body, 0)

        def _merge(i, _):
            acc0[pl.ds(i * 16, 16)] = (acc0[pl.ds(i * 16, 16)]
                                       + acc1[pl.ds(i * 16, 16)])
            return 0

        lax.fori_loop(0, HW // 16, _merge, 0)

        base = (C + q) * HW
        pltpu.sync_copy(acc0, out_hbm.at[pl.ds(base, HW)])

    for slot in range(2):
        do_feat(slot * NWORKERS + wid)

    @pl.when(wid >= NWORKERS - 4)
    def _dens():
        do_dens(wid - (NWORKERS - 4))


def _run_splat(packed8, feat2):
    mesh = plsc.VectorSubcoreMesh(core_axis_name="c", subcore_axis_name="s",
                                  num_cores=NCORES, num_subcores=NSUB)
    f = functools.partial(
        pl.kernel,
        out_type=jax.ShapeDtypeStruct((NROW * HW,), jnp.float32),
        mesh=mesh,
        compiler_params=pltpu.CompilerParams(needs_layout_passes=False),
        scratch_types=[
            pltpu.VMEM((HW,), jnp.float32),
            pltpu.VMEM((HW,), jnp.float32),
            pltpu.VMEM((2, 8, S), jnp.float32),
            pltpu.VMEM((2, 2, S), jnp.float32),
            pltpu.SemaphoreType.DMA((2,)),
        ],
    )(_splat_body)
    return f(packed8, feat2)


HWD = HW // 8


def _norm_body(splat_ref, out_ref):
    s = splat_ref[0]
    n = s[C:C + 1] + s[C + 1:C + 2] + s[C + 2:C + 3] + s[C + 3:C + 4]
    out_ref[0] = s[0:C] / jnp.maximum(n, 1e-7)


def _run_norm(splat):
    return pl.pallas_call(
        _norm_body,
        grid=(1, 8),
        in_specs=[pl.BlockSpec((1, NROW, HWD), lambda b, j: (b, 0, j))],
        out_specs=pl.BlockSpec((1, C, HWD), lambda b, j: (b, 0, j)),
        out_shape=jax.ShapeDtypeStruct((1, C, HW), jnp.float32),
    )(splat)


def kernel(feat_ref, flow, W1, b1, W2, b2):
    featp = jnp.pad(jnp.transpose(feat_ref, (0, 2, 3, 1)),
                    ((0, 0), (1, 1), (1, 1), (0, 0)))
    w1r = jnp.transpose(W1, (2, 3, 1, 0)).reshape(9, C, CO1)
    b1r = b1.reshape(1, CO1)
    w2r = jnp.pad(jnp.transpose(W2, (2, 3, 1, 0)).reshape(9, CO1, 1),
                  ((0, 0), (0, 0), (0, 7)))
    b2r = jnp.pad(b2.reshape(1, 1), ((0, 0), (0, 7)))

    feat2 = feat_ref.reshape(B, C, HW)
    warped_parts = []
    metric_parts = []
    for b in range(B):
        metric8 = _run_metric(featp[b:b + 1], w1r, b1r, w2r, b2r)
        metric2d = metric8[..., 0]
        metric_parts.append(metric2d)
        wm, idxo = _run_taps(metric2d, flow[b:b + 1])
        packed8 = jnp.concatenate(
            [wm.reshape(1, 4, HW), idxo.reshape(1, 4, HW)], axis=1)
        splat = _run_splat(packed8, feat2[b:b + 1]).reshape(1, NROW, HW)
        warped_parts.append(_run_norm(splat))
    warped = jnp.concatenate(warped_parts, axis=0).reshape(B, C, H, W)
    metric = jnp.concatenate(metric_parts, axis=0)[:, None]
    return warped, metric


# speedup vs baseline: 3.1236x; 1.0121x over previous
import functools

import jax
import jax.numpy as jnp
from jax import lax
from jax.experimental import pallas as pl
from jax.experimental.pallas import tpu as pltpu
from jax.experimental.pallas import tpu_sc as plsc

B, C, H, W = 4, 128, 224, 224
HW = H * W
CO1 = 64
RB = 32
NB = H // RB

NCORES, NSUB, LANES = 2, 16, 16
NWORKERS = NCORES * NSUB
NCHUNK = C // 2 + 1
NTASK = B * NCHUNK
NSLOT = (NTASK + NWORKERS - 1) // NWORKERS
S = 1024
NSLICES = HW // S
GROUPS = S // LANES


def _metric_body(feat_hbm, w1_ref, b1_ref, w2_ref, b2_ref, out_ref,
                 h_pad, in_buf, sem):
    bb = pl.program_id(0)
    i = pl.program_id(1)
    g = bb * NB + i

    @pl.when(i == 0)
    def _zero():
        h_pad[...] = jnp.zeros((H + 2, W + 2, CO1), jnp.float32)

    def copy_band(gg, k):
        bn = lax.div(gg, NB)
        bi = gg - bn * NB
        return pltpu.make_async_copy(
            feat_hbm.at[bn, pl.ds(bi * RB, RB + 2)], in_buf.at[k], sem.at[k])

    @pl.when(jnp.logical_and(bb == 0, i == 0))
    def _prime():
        copy_band(0, 0).start()

    @pl.when(i < NB)
    def _conv1():
        k = lax.rem(g, 2)
        copy_band(g, k).wait()

        @pl.when(g + 1 < pl.num_programs(0) * NB)
        def _prefetch():
            copy_band(g + 1, 1 - k).start()

        acc = jnp.zeros((RB * W, CO1), jnp.float32)
        for ky in range(3):
            for kx in range(3):
                x = in_buf[k, pl.ds(ky, RB), pl.ds(kx, W), :]
                acc += jnp.dot(x.reshape(RB * W, C), w1_ref[3 * ky + kx],
                               preferred_element_type=jnp.float32)
        acc += b1_ref[...]
        h = acc * jax.nn.sigmoid(acc)
        h_pad[pl.ds(i * RB + 1, RB), pl.ds(1, W), :] = h.reshape(RB, W, CO1)

    @pl.when(i >= 1)
    def _conv2():
        y0 = (i - 1) * RB
        macc = jnp.zeros((RB * W, 8), jnp.float32)
        for ky in range(3):
            for kx in range(3):
                xs = h_pad[pl.ds(y0 + ky, RB), pl.ds(kx, W), :]
                macc += jnp.dot(xs.reshape(RB * W, CO1), w2_ref[3 * ky + kx],
                                preferred_element_type=jnp.float32)
        out_ref[0] = (macc + b2_ref[...]).reshape(RB, W, 8)


def _run_metric(featp, w1r, b1r, w2r, b2r):
    Bk = featp.shape[0]
    return pl.pallas_call(
        _metric_body,
        grid=(Bk, NB + 1),
        in_specs=[
            pl.BlockSpec(memory_space=pl.ANY),
            pl.BlockSpec((9, C, CO1), lambda b, i: (0, 0, 0)),
            pl.BlockSpec((1, CO1), lambda b, i: (0, 0)),
            pl.BlockSpec((9, CO1, 8), lambda b, i: (0, 0, 0)),
            pl.BlockSpec((1, 8), lambda b, i: (0, 0)),
        ],
        out_specs=pl.BlockSpec(
            (1, RB, W, 8),
            lambda b, i: (b, jnp.clip(i - 1, 0, NB - 1), 0, 0)),
        out_shape=jax.ShapeDtypeStruct((Bk, H, W, 8), jnp.float32),
        scratch_shapes=[
            pltpu.VMEM((H + 2, W + 2, CO1), jnp.float32),
            pltpu.VMEM((2, RB + 2, W + 2, C), jnp.float32),
            pltpu.SemaphoreType.DMA((2,)),
        ],
    )(featp, w1r, b1r, w2r, b2r)


def _taps_body(metric_ref, flow_ref, wm_ref, idx_ref):
    xg = lax.broadcasted_iota(jnp.int32, (H, W), 1).astype(jnp.float32)
    yg = lax.broadcasted_iota(jnp.int32, (H, W), 0).astype(jnp.float32)
    gx = xg + flow_ref[0, 0]
    gy = yg + flow_ref[0, 1]
    x0 = jnp.floor(gx)
    y0 = jnp.floor(gy)
    m = jnp.exp(metric_ref[0])
    t = 0
    for dy in (0.0, 1.0):
        for dx in (0.0, 1.0):
            xi = x0 + dx
            yi = y0 + dy
            w = (1.0 - jnp.abs(gx - xi)) * (1.0 - jnp.abs(gy - yi))
            valid = (xi >= 0.0) & (xi <= W - 1.0) & (yi >= 0.0) & (yi <= H - 1.0)
            wv = jnp.where(valid, w, 0.0) * m
            xc = jnp.clip(xi, 0.0, W - 1.0).astype(jnp.int32)
            yc = jnp.clip(yi, 0.0, H - 1.0).astype(jnp.int32)
            idx = jnp.clip(yc * W + xc, 0, HW - 1).astype(jnp.float32)
            wm_ref[0, t] = wv
            idx_ref[0, t] = idx
            t += 1


def _run_taps(metric2d, flow):
    Bk = metric2d.shape[0]
    return pl.pallas_call(
        _taps_body,
        grid=(Bk,),
        in_specs=[
            pl.BlockSpec((1, H, W), lambda b: (b, 0, 0)),
            pl.BlockSpec((1, 2, H, W), lambda b: (b, 0, 0, 0)),
        ],
        out_specs=[
            pl.BlockSpec((1, 4, H, W), lambda b: (b, 0, 0, 0)),
            pl.BlockSpec((1, 4, H, W), lambda b: (b, 0, 0, 0)),
        ],
        out_shape=[
            jax.ShapeDtypeStruct((Bk, 4, H, W), jnp.float32),
            jax.ShapeDtypeStruct((Bk, 4, H, W), jnp.float32),
        ],
    )(metric2d, flow)


NROW = C + 4
QSL = NSLICES // 4


def _splat_body(pk_hbm, feat_hbm, out_hbm, acc0, acc1, pk, vl, sems):
    wid = lax.axis_index("s") * NCORES + lax.axis_index("c")

    def issue(chunk, si, k, with_vals):
        off = si * S
        pltpu.async_copy(pk_hbm.at[0, :, pl.ds(off, S)], pk.at[k], sems.at[k])
        if with_vals:
            pltpu.async_copy(
                feat_hbm.at[0, pl.ds(chunk * 2, 2), pl.ds(off, S)],
                vl.at[k], sems.at[k])

    def drain(k, with_vals):
        pltpu.make_async_copy(pk_hbm.at[0, :, pl.ds(0, S)], pk.at[k],
                              sems.at[k]).wait()
        if with_vals:
            pltpu.make_async_copy(feat_hbm.at[0, pl.ds(0, 2), pl.ds(0, S)],
                                  vl.at[k], sems.at[k]).wait()

    def zero_acc(both):
        def _zero(i, _):
            for j in range(4):
                acc0[pl.ds(i * 64 + j * 16, 16)] = jnp.zeros((16,), jnp.float32)
                if both:
                    acc1[pl.ds(i * 64 + j * 16, 16)] = jnp.zeros((16,),
                                                                 jnp.float32)
            return 0

        lax.fori_loop(0, HW // 64, _zero, 0)

    def do_feat(chunk):
        issue(chunk, 0, 0, True)
        zero_acc(True)

        def slice_body(si, _):
            k = lax.rem(si, 2)

            @pl.when(si + 1 < NSLICES)
            def _pref():
                issue(chunk, si + 1, 1 - k, True)

            drain(k, True)

            def grp(g, _):
                g0 = g * LANES
                v0 = vl[k, 0, pl.ds(g0, LANES)]
                v1 = vl[k, 1, pl.ds(g0, LANES)]
                for t in range(4):
                    w = pk[k, t, pl.ds(g0, LANES)]
                    ix = pk[k, 4 + t, pl.ds(g0, LANES)].astype(jnp.int32)
                    plsc.addupdate_scatter(acc0, [ix], v0 * w)
                    plsc.addupdate_scatter(acc1, [ix], v1 * w)
                return 0

            lax.fori_loop(0, GROUPS, grp, 0)
            return 0

        lax.fori_loop(0, NSLICES, slice_body, 0)
        base = chunk * 2 * HW
        pltpu.sync_copy(acc0, out_hbm.at[pl.ds(base, HW)])
        pltpu.sync_copy(acc1, out_hbm.at[pl.ds(base + HW, HW)])

    def do_dens(q):
        s_lo = q * QSL
        s_hi = jnp.where(q == 3, NSLICES, s_lo + QSL)
        issue(0, s_lo, 0, False)
        zero_acc(True)

        def slice_body(si, _):
            k = lax.rem(si - s_lo, 2)

            @pl.when(si + 1 < s_hi)
            def _pref():
                issue(0, si + 1, 1 - k, False)

            drain(k, False)

            def grp(g, _):
                g0 = g * LANES
                for t in range(4):
                    w = pk[k, t, pl.ds(g0, LANES)]
                    ix = pk[k, 4 + t, pl.ds(g0, LANES)].astype(jnp.int32)
                    plsc.addupdate_scatter(acc0 if t < 2 else acc1, [ix], w)
                return 0

            lax.fori_loop(0, GROUPS, grp, 0)
            return 0

        lax.fori_loop(s_lo, s_hi, slice_body, 0)

        def _merge(i, _):
            acc0[pl.ds(i * 16, 16)] = (acc0[pl.ds(i * 16, 16)]
                                       + acc1[pl.ds(i * 16, 16)])
            return 0

        lax.fori_loop(0, HW // 16, _merge, 0)

        base = (C + q) * HW
        pltpu.sync_copy(acc0, out_hbm.at[pl.ds(base, HW)])

    for slot in range(2):
        do_feat(slot * NWORKERS + wid)

    @pl.when(wid >= NWORKERS - 4)
    def _dens():
        do_dens(wid - (NWORKERS - 4))


def _run_splat(packed8, feat2):
    mesh = plsc.VectorSubcoreMesh(core_axis_name="c", subcore_axis_name="s",
                                  num_cores=NCORES, num_subcores=NSUB)
    f = functools.partial(
        pl.kernel,
        out_type=jax.ShapeDtypeStruct((NROW * HW,), jnp.float32),
        mesh=mesh,
        compiler_params=pltpu.CompilerParams(needs_layout_passes=False),
        scratch_types=[
            pltpu.VMEM((HW,), jnp.float32),
            pltpu.VMEM((HW,), jnp.float32),
            pltpu.VMEM((2, 8, S), jnp.float32),
            pltpu.VMEM((2, 2, S), jnp.float32),
            pltpu.SemaphoreType.DMA((2,)),
        ],
    )(_splat_body)
    return f(packed8, feat2)


HWD = HW // 8


def _norm_body(splat_ref, out_ref):
    s = splat_ref[0]
    n = s[C:C + 1] + s[C + 1:C + 2] + s[C + 2:C + 3] + s[C + 3:C + 4]
    out_ref[0] = s[0:C] / jnp.maximum(n, 1e-7)


def _run_norm(splat):
    return pl.pallas_call(
        _norm_body,
        grid=(1, 8),
        in_specs=[pl.BlockSpec((1, NROW, HWD), lambda b, j: (b, 0, j))],
        out_specs=pl.BlockSpec((1, C, HWD), lambda b, j: (b, 0, j)),
        out_shape=jax.ShapeDtypeStruct((1, C, HW), jnp.float32),
    )(splat)


def kernel(feat_ref, flow, W1, b1, W2, b2):
    featp = jnp.pad(jnp.transpose(feat_ref, (0, 2, 3, 1)),
                    ((0, 0), (1, 1), (1, 1), (0, 0)))
    w1r = jnp.transpose(W1, (2, 3, 1, 0)).reshape(9, C, CO1)
    b1r = b1.reshape(1, CO1)
    w2r = jnp.pad(jnp.transpose(W2, (2, 3, 1, 0)).reshape(9, CO1, 1),
                  ((0, 0), (0, 0), (0, 7)))
    b2r = jnp.pad(b2.reshape(1, 1), ((0, 0), (0, 7)))

    feat2 = feat_ref.reshape(B, C, HW)
    warped_parts = []
    metric_parts = []
    prev = None
    for b in range(B):
        metric8 = _run_metric(featp[b:b + 1], w1r, b1r, w2r, b2r)
        metric2d = metric8[..., 0]
        metric_parts.append(metric2d)
        wm, idxo = _run_taps(metric2d, flow[b:b + 1])
        packed8 = jnp.concatenate(
            [wm.reshape(1, 4, HW), idxo.reshape(1, 4, HW)], axis=1)
        if prev is not None:
            packed8, _ = lax.optimization_barrier((packed8, prev))
        splat = _run_splat(packed8, feat2[b:b + 1])
        prev = splat
        splat = splat.reshape(1, NROW, HW)
        warped_parts.append(_run_norm(splat))
    warped = jnp.concatenate(warped_parts, axis=0).reshape(B, C, H, W)
    metric = jnp.concatenate(metric_parts, axis=0)[:, None]
    return warped, metric
